# baseline (speedup 1.0000x reference)
import functools

import jax
import jax.numpy as jnp
from jax import lax
from jax.experimental import pallas as pl
from jax.experimental.pallas import tpu as pltpu
from jax.experimental.pallas import tpu_sc as plsc

N_NODES = 2048
N_EDGES = 65536
TILE = 256
N_TILES = N_EDGES // TILE
CH = 32

N_WORKERS = 32
EPW = N_EDGES // N_WORKERS
SUB = 128
N_SUB = EPW // SUB

NEG_INF = float("-inf")


def _proj_kernel(x_ref, wa_ref, bias_ref, wb_ref, pp_ref, qt_ref):
    f = wa_ref.shape[1]
    pp_ref[pl.ds(0, N_NODES), :] = (
        jnp.dot(x_ref[...], wa_ref[...], preferred_element_type=jnp.float32)
        + bias_ref[...]
    )
    pp_ref[pl.ds(N_NODES, TILE), :] = jnp.zeros((TILE, f), jnp.float32)
    qt_ref[:, pl.ds(0, f)] = jnp.dot(
        x_ref[...], wb_ref[...], preferred_element_type=jnp.float32
    )
    qt_ref[:, pl.ds(f, f)] = jnp.zeros((N_NODES, f), jnp.float32)


def _project(x, wa, bias, wb):
    f = wa.shape[1]
    return pl.pallas_call(
        _proj_kernel,
        out_shape=(
            jax.ShapeDtypeStruct((N_NODES + TILE, f), jnp.float32),
            jax.ShapeDtypeStruct((N_NODES, 2 * f), jnp.float32),
        ),
    )(x, wa, bias, wb)


NL = 16
EPL = EPW // NL


def _sc_hist_body(dst_hbm, zeros_hbm, counts_hbm, lh_hbm, dstv, lanehist, stage):
    wid = lax.axis_index("s") * 2 + lax.axis_index("c")
    base = wid * EPW
    pltpu.sync_copy(dst_hbm.at[pl.ds(base, EPW)], dstv)
    pltpu.sync_copy(zeros_hbm, lanehist)
    zero = jnp.zeros((NL,), jnp.int32)
    lane = lax.iota(jnp.int32, NL)

    def hbody(k, c):
        d = plsc.load_gather(dstv, [lane * EPL + k])
        a = lane * N_NODES + d
        cur = plsc.load_gather(lanehist, [a])
        plsc.store_scatter(lanehist, [a], cur + 1)
        return c

    lax.fori_loop(0, EPL, hbody, 0)
    pltpu.sync_copy(lanehist, lh_hbm.at[wid])

    def rbody(v4, c):
        for u in range(4):
            v = v4 * 4 + u
            acc = zero
            for l in range(NL):
                acc = acc + lanehist[pl.ds(l * N_NODES + v * NL, NL)]
            stage[pl.ds(v * NL, NL)] = acc
        return c

    lax.fori_loop(0, N_NODES // NL // 4, rbody, 0)
    pltpu.sync_copy(stage, counts_hbm.at[wid])


def _sc_wprefix_body(counts_hbm, wstart_hbm, totals_hbm, cbuf, obuf, tot):
    wid = lax.axis_index("s") * 2 + lax.axis_index("c")
    npw = N_NODES // N_WORKERS
    c0 = wid * npw
    for w in range(N_WORKERS):
        pltpu.sync_copy(counts_hbm.at[w, pl.ds(c0, npw)], cbuf.at[pl.ds(w * npw, npw)])
    for j in range(npw // NL):
        run = jnp.zeros((NL,), jnp.int32)
        for w in range(N_WORKERS):
            c = cbuf[pl.ds(w * npw + j * NL, NL)]
            obuf[pl.ds(w * npw + j * NL, NL)] = run
            run = run + c
        tot[pl.ds(j * NL, NL)] = run
    for w in range(N_WORKERS):
        pltpu.sync_copy(obuf.at[pl.ds(w * npw, npw)], wstart_hbm.at[w, pl.ds(c0, npw)])
    pltpu.sync_copy(tot, totals_hbm.at[pl.ds(c0, npw)])


def _sc_rowptr_body(totals_hbm, rp_hbm, tbuf, rbuf):
    wid = lax.axis_index("s") * 2 + lax.axis_index("c")

    @pl.when(wid == 0)
    def _():
        pltpu.sync_copy(totals_hbm, tbuf)

        def body(j, carry):
            v = tbuf[pl.ds(j * NL, NL)]
            s = plsc.cumsum(v)
            rbuf[pl.ds(j * NL, NL)] = (s - v) + carry
            return carry + jnp.sum(v)

        total = lax.fori_loop(0, N_NODES // NL, body, jnp.int32(0))
        tailv = rbuf[pl.ds(N_NODES - 8, NL)]
        sel = lax.iota(jnp.int32, NL) >= 8
        rbuf[pl.ds(N_NODES - 8, NL)] = jnp.where(sel, total, tailv)
        pltpu.sync_copy(rbuf, rp_hbm)


def _sc_scatter_body(dst_hbm, src_hbm, wstart_hbm, rp_hbm, lh_hbm, sd_hbm, ss_hbm,
                     dstv, srcv, lanehist, postab, wsbuf, rpbuf, posb, sem):
    wid = lax.axis_index("s") * 2 + lax.axis_index("c")
    base = wid * EPW
    pltpu.sync_copy(dst_hbm.at[pl.ds(base, EPW)], dstv)
    pltpu.sync_copy(src_hbm.at[pl.ds(base, EPW)], srcv)
    pltpu.sync_copy(wstart_hbm.at[wid], wsbuf)
    pltpu.sync_copy(rp_hbm.at[pl.ds(0, N_NODES)], rpbuf)
    pltpu.sync_copy(lh_hbm.at[wid], lanehist)
    lane = lax.iota(jnp.int32, NL)

    def ibody(j, c):
        run = rpbuf[pl.ds(j * NL, NL)] + wsbuf[pl.ds(j * NL, NL)]
        for l in range(NL):
            cnt = lanehist[pl.ds(l * N_NODES + j * NL, NL)]
            postab[pl.ds(l * N_NODES + j * NL, NL)] = run
            run = run + cnt
        return c

    lax.fori_loop(0, N_NODES // NL, ibody, 0)

    def sbody(k, c):
        e = lane * EPL + k
        d = plsc.load_gather(dstv, [e])
        a = lane * N_NODES + d
        p = plsc.load_gather(postab, [a])
        plsc.store_scatter(postab, [a], p + 1)
        plsc.store_scatter(posb, [lane, jnp.full((NL,), k, jnp.int32)], p)
        return c

    lax.fori_loop(0, EPL, sbody, 0)

    for l in range(NL):
        cd = pltpu.async_copy(dstv.at[pl.ds(l * EPL, EPL)], sd_hbm.at[posb.at[l]], sem)
        cd.wait()
        cs = pltpu.async_copy(srcv.at[pl.ds(l * EPL, EPL)], ss_hbm.at[posb.at[l]], sem)
        cs.wait()


def _sc_sort(dst, src):
    mesh = plsc.VectorSubcoreMesh(core_axis_name="c", subcore_axis_name="s")
    nolayout = pltpu.CompilerParams(needs_layout_passes=False)
    zeros32 = jnp.zeros((NL * N_NODES,), jnp.int32)
    counts, lh = functools.partial(
        pl.kernel, _sc_hist_body, mesh=mesh, compiler_params=nolayout,
        out_type=(
            jax.ShapeDtypeStruct((N_WORKERS, N_NODES), jnp.int32),
            jax.ShapeDtypeStruct((N_WORKERS, NL * N_NODES), jnp.int32),
        ),
        scratch_types=[
            pltpu.VMEM((EPW,), jnp.int32),
            pltpu.VMEM((NL * N_NODES,), jnp.int32),
            pltpu.VMEM((N_NODES,), jnp.int32),
        ],
    )()(dst, zeros32)
    wstart, totals = functools.partial(
        pl.kernel, _sc_wprefix_body, mesh=mesh, compiler_params=nolayout,
        out_type=(
            jax.ShapeDtypeStruct((N_WORKERS, N_NODES), jnp.int32),
            jax.ShapeDtypeStruct((N_NODES,), jnp.int32),
        ),
        scratch_types=[
            pltpu.VMEM((N_NODES,), jnp.int32),
            pltpu.VMEM((N_NODES,), jnp.int32),
            pltpu.VMEM((N_NODES // N_WORKERS,), jnp.int32),
        ],
    )()(counts)
    rp = functools.partial(
        pl.kernel, _sc_rowptr_body, mesh=mesh, compiler_params=nolayout,
        out_type=jax.ShapeDtypeStruct((N_NODES + 8,), jnp.int32),
        scratch_types=[
            pltpu.VMEM((N_NODES,), jnp.int32),
            pltpu.VMEM((N_NODES + 8,), jnp.int32),
        ],
    )()(totals)
    sd, ssrc = functools.partial(
        pl.kernel, _sc_scatter_body, mesh=mesh, compiler_params=nolayout,
        out_type=(
            jax.ShapeDtypeStruct((N_EDGES,), jnp.int32),
            jax.ShapeDtypeStruct((N_EDGES,), jnp.int32),
        ),
        scratch_types=[
            pltpu.VMEM((EPW,), jnp.int32),
            pltpu.VMEM((EPW,), jnp.int32),
            pltpu.VMEM((NL * N_NODES,), jnp.int32),
            pltpu.VMEM((NL * N_NODES,), jnp.int32),
            pltpu.VMEM((N_NODES,), jnp.int32),
            pltpu.VMEM((N_NODES,), jnp.int32),
            pltpu.VMEM((NL, EPL), jnp.int32),
            pltpu.SemaphoreType.DMA,
        ],
    )()(dst, src, wstart, rp, lh)
    return sd, ssrc, rp


RING = 4


def _sc_gather_body(qt_hbm, ss_hbm, outq_hbm,
                    ss_v, qr0, qr1, qr2, qr3, sq0, sq1, sq2, sq3):
    wid = lax.axis_index("s") * 2 + lax.axis_index("c")
    base = wid * EPW
    pltpu.sync_copy(ss_hbm.at[pl.ds(base, EPW)], ss_v)
    qrb = [qr0, qr1, qr2, qr3]
    sqs = [sq0, sq1, sq2, sq3]

    def fire(j):
        r = j % RING
        idx_s = ss_v.at[pl.ds(j * SUB, SUB)]
        return pltpu.async_copy(qt_hbm.at[idx_s], qrb[r], sqs[r])

    inflight = [fire(j) for j in range(RING)]
    for j in range(N_SUB):
        r = j % RING
        inflight[r].wait()
        off = base + j * SUB
        pltpu.sync_copy(qrb[r], outq_hbm.at[pl.ds(off, SUB)])
        if j + RING < N_SUB:
            inflight[r] = fire(j + RING)


def _gather_q(qtab, ssrc):
    f2 = qtab.shape[1]
    mesh = plsc.VectorSubcoreMesh(core_axis_name="c", subcore_axis_name="s")
    fn = functools.partial(
        pl.kernel,
        _sc_gather_body,
        mesh=mesh,
        out_type=jax.ShapeDtypeStruct((N_EDGES, f2), jnp.float32),
        scratch_types=[
            pltpu.VMEM((EPW,), jnp.int32),
        ]
        + [pltpu.VMEM((SUB, f2), jnp.float32) for _ in range(RING)]
        + [pltpu.SemaphoreType.DMA for _ in range(RING)],
    )()
    return fn(qtab, ssrc)


def _conv_kernel(rp_ref, nf_ref, nl_ref, sd_ref, q_ref, p_ref, w_ref, b_ref,
                 o_ref, h_ref, m_ref, acc_ref):
    t = pl.program_id(0)
    c = o_ref.shape[1]
    f = h_ref.shape[1]
    tbase = t * TILE
    nf = nf_ref[t]
    nl = nl_ref[t]

    @pl.when(t == 0)
    def _init():
        acc_ref[...] = jnp.full(acc_ref.shape, NEG_INF, jnp.float32)
        m_ref[pl.ds(TILE, CH), :] = jnp.full((CH, c), NEG_INF, jnp.float32)

    iota_ch = lax.broadcasted_iota(jnp.int32, (CH, 1), 0)

    sdv = sd_ref[...].reshape(1, TILE)
    riota = lax.broadcasted_iota(jnp.int32, (TILE, TILE), 0)
    st = jnp.where(riota + nf == sdv, 1.0, 0.0)
    pwin = p_ref[pl.ds(nf, TILE), :]
    prow = lax.dot_general(
        st, pwin, (((0,), (0,)), ((), ())),
        precision=lax.Precision.HIGHEST,
        preferred_element_type=jnp.float32,
    )
    h_ref[...] = jnp.maximum(prow + q_ref[:, :f], 0.0)

    @pl.when(nl - nf > TILE - 1)
    def _fb():
        def fb_node(n, carry):
            s = jnp.maximum(rp_ref[n] - tbase, 0)
            e = jnp.minimum(rp_ref[n + 1] - tbase, TILE)
            base0 = jnp.minimum((s // 8) * 8, TILE - CH)
            nch = (e - base0 + CH - 1) // CH

            def fb_chunk(ci, carry2):
                b0 = jnp.minimum(base0 + ci * CH, TILE - CH)
                ridx = b0 + iota_ch
                mask = (ridx >= s) & (ridx < e)
                pn = p_ref[pl.ds(n, 1), :]
                newv = jnp.maximum(pn + q_ref[pl.ds(b0, CH), :f], 0.0)
                h_ref[pl.ds(b0, CH), :] = jnp.where(
                    mask, newv, h_ref[pl.ds(b0, CH), :]
                )
                return carry2

            lax.fori_loop(0, nch, fb_chunk, 0)
            return carry

        lax.fori_loop(nf + TILE, nl + 1, fb_node, 0)

    m_ref[pl.ds(0, TILE), :] = (
        jnp.dot(h_ref[...], w_ref[...], preferred_element_type=jnp.float32)
        + b_ref[...]
    )

    def node_body(n, carry):
        s = jnp.maximum(rp_ref[n] - tbase, 0)
        e = jnp.minimum(rp_ref[n + 1] - tbase, TILE)
        base0 = (s // 8) * 8
        nch = (e - base0 + CH - 1) // CH

        def chunk_body(ci, red):
            b0 = base0 + ci * CH
            rows = m_ref[pl.ds(b0, CH), :]
            ridx = b0 + iota_ch
            mask = (ridx >= s) & (ridx < e)
            return jnp.maximum(red, jnp.where(mask, rows, NEG_INF))

        red = lax.fori_loop(
            0, nch, chunk_body, jnp.full((CH, c), NEG_INF, jnp.float32)
        )
        rowmax = jnp.max(red, axis=0, keepdims=True)
        acc_ref[pl.ds(n, 1), :] = jnp.maximum(acc_ref[pl.ds(n, 1), :], rowmax)
        return carry

    lax.fori_loop(nf, nl + 1, node_body, 0)

    @pl.when(t == N_TILES - 1)
    def _fin():
        a = acc_ref[...]
        o_ref[...] = jnp.where(jnp.isneginf(a), 0.0, a)


def _edge_conv(rp, nfirst, nlast, sd3, rows_q, ppad, w_t, b_row):
    c = w_t.shape[1]
    f2 = rows_q.shape[1]
    grid_spec = pltpu.PrefetchScalarGridSpec(
        num_scalar_prefetch=3,
        grid=(N_TILES,),
        in_specs=[
            pl.BlockSpec((1, 1, TILE), lambda t, *_: (t, 0, 0)),
            pl.BlockSpec((TILE, f2), lambda t, *_: (t, 0)),
            pl.BlockSpec(ppad.shape, lambda t, *_: (0, 0)),
            pl.BlockSpec(w_t.shape, lambda t, *_: (0, 0)),
            pl.BlockSpec(b_row.shape, lambda t, *_: (0, 0)),
        ],
        out_specs=pl.BlockSpec((N_NODES, c), lambda t, *_: (0, 0)),
        scratch_shapes=[
            pltpu.VMEM((TILE, f2 // 2), jnp.float32),
            pltpu.VMEM((TILE + CH, c), jnp.float32),
            pltpu.VMEM((N_NODES, c), jnp.float32),
        ],
    )
    return pl.pallas_call(
        _conv_kernel,
        grid_spec=grid_spec,
        out_shape=jax.ShapeDtypeStruct((N_NODES, c), jnp.float32),
    )(rp, nfirst, nlast, sd3, rows_q, ppad, w_t, b_row)


def _readout_kernel(h_ref, w_ref, b_ref, o_ref):
    o_ref[...] = (
        lax.dot_general(
            h_ref[...],
            w_ref[...],
            (((0,), (1,)), ((), ())),
            preferred_element_type=jnp.float32,
        )
        + b_ref[...]
    )


def kernel(x, edge_index, W1, b1, W2, b2, W3, b3, W4, b4, Wr, br):
    in_ch = x.shape[1]
    hid = W2.shape[0]
    src = edge_index[0].astype(jnp.int32)
    dst = edge_index[1].astype(jnp.int32)

    sd, ssrc, rp = _sc_sort(dst, src)
    sd2 = sd.reshape(N_TILES, TILE)
    nfirst = sd2[:, 0]
    nlast = sd2[:, -1]
    sd3 = sd.reshape(N_TILES, 1, TILE)

    pp1, qt1 = _project(x, W1[:, :in_ch].T, b1.reshape(1, -1), W1[:, in_ch:].T)
    rq1 = _gather_q(qt1, ssrc)
    h1 = _edge_conv(rp, nfirst, nlast, sd3, rq1, pp1, W2.T, b2.reshape(1, -1))

    pp2, qt2 = _project(h1, W3[:, :hid].T, b3.reshape(1, -1), W3[:, hid:].T)
    rq2 = _gather_q(qt2, ssrc)
    h2 = _edge_conv(rp, nfirst, nlast, sd3, rq2, pp2, W4.T, b4.reshape(1, -1))

    out = pl.pallas_call(
        _readout_kernel,
        out_shape=jax.ShapeDtypeStruct((Wr.shape[0], Wr.shape[0]), jnp.float32),
    )(h2, Wr, br.reshape(1, -1))
    return out


# speedup vs baseline: 1.0879x; 1.0879x over previous
import functools

import jax
import jax.numpy as jnp
from jax import lax
from jax.experimental import pallas as pl
from jax.experimental.pallas import tpu as pltpu
from jax.experimental.pallas import tpu_sc as plsc

N_NODES = 2048
N_EDGES = 65536
TILE = 256
N_TILES = N_EDGES // TILE
CH = 32
PWIN = 64

N_WORKERS = 32
EPW = N_EDGES // N_WORKERS
SUB = 128
N_SUB = EPW // SUB

NEG_INF = float("-inf")


def _proj_kernel(x_ref, wa_ref, bias_ref, wb_ref, pp_ref, qt_ref):
    f = wa_ref.shape[1]
    pp_ref[pl.ds(0, N_NODES), :] = (
        jnp.dot(x_ref[...], wa_ref[...], preferred_element_type=jnp.float32)
        + bias_ref[...]
    )
    pp_ref[pl.ds(N_NODES, TILE), :] = jnp.zeros((TILE, f), jnp.float32)
    qt_ref[:, pl.ds(0, f)] = jnp.dot(
        x_ref[...], wb_ref[...], preferred_element_type=jnp.float32
    )
    qt_ref[:, pl.ds(f, f)] = jnp.zeros((N_NODES, f), jnp.float32)


def _project(x, wa, bias, wb):
    f = wa.shape[1]
    return pl.pallas_call(
        _proj_kernel,
        out_shape=(
            jax.ShapeDtypeStruct((N_NODES + TILE, f), jnp.float32),
            jax.ShapeDtypeStruct((N_NODES, 2 * f), jnp.float32),
        ),
    )(x, wa, bias, wb)


NL = 16
EPL = EPW // NL


def _sc_hist_body(dst_hbm, zeros_hbm, counts_hbm, lh_hbm, dstv, lanehist, stage):
    wid = lax.axis_index("s") * 2 + lax.axis_index("c")
    base = wid * EPW
    pltpu.sync_copy(dst_hbm.at[pl.ds(base, EPW)], dstv)
    pltpu.sync_copy(zeros_hbm, lanehist)
    zero = jnp.zeros((NL,), jnp.int32)
    lane = lax.iota(jnp.int32, NL)

    def hbody(k, c):
        d = plsc.load_gather(dstv, [lane * EPL + k])
        a = lane * N_NODES + d
        cur = plsc.load_gather(lanehist, [a])
        plsc.store_scatter(lanehist, [a], cur + 1)
        return c

    lax.fori_loop(0, EPL, hbody, 0)
    pltpu.sync_copy(lanehist, lh_hbm.at[wid])

    def rbody(v4, c):
        for u in range(4):
            v = v4 * 4 + u
            acc = zero
            for l in range(NL):
                acc = acc + lanehist[pl.ds(l * N_NODES + v * NL, NL)]
            stage[pl.ds(v * NL, NL)] = acc
        return c

    lax.fori_loop(0, N_NODES // NL // 4, rbody, 0)
    pltpu.sync_copy(stage, counts_hbm.at[wid])


def _sc_wprefix_body(counts_hbm, wstart_hbm, totals_hbm, cbuf, obuf, tot):
    wid = lax.axis_index("s") * 2 + lax.axis_index("c")
    npw = N_NODES // N_WORKERS
    c0 = wid * npw
    for w in range(N_WORKERS):
        pltpu.sync_copy(counts_hbm.at[w, pl.ds(c0, npw)], cbuf.at[pl.ds(w * npw, npw)])
    for j in range(npw // NL):
        run = jnp.zeros((NL,), jnp.int32)
        for w in range(N_WORKERS):
            c = cbuf[pl.ds(w * npw + j * NL, NL)]
            obuf[pl.ds(w * npw + j * NL, NL)] = run
            run = run + c
        tot[pl.ds(j * NL, NL)] = run
    for w in range(N_WORKERS):
        pltpu.sync_copy(obuf.at[pl.ds(w * npw, npw)], wstart_hbm.at[w, pl.ds(c0, npw)])
    pltpu.sync_copy(tot, totals_hbm.at[pl.ds(c0, npw)])


def _sc_rowptr_body(totals_hbm, rp_hbm, tbuf, rbuf):
    wid = lax.axis_index("s") * 2 + lax.axis_index("c")

    @pl.when(wid == 0)
    def _():
        pltpu.sync_copy(totals_hbm, tbuf)

        def body(j, carry):
            v = tbuf[pl.ds(j * NL, NL)]
            s = plsc.cumsum(v)
            rbuf[pl.ds(j * NL, NL)] = (s - v) + carry
            return carry + jnp.sum(v)

        total = lax.fori_loop(0, N_NODES // NL, body, jnp.int32(0))
        tailv = rbuf[pl.ds(N_NODES - 8, NL)]
        sel = lax.iota(jnp.int32, NL) >= 8
        rbuf[pl.ds(N_NODES - 8, NL)] = jnp.where(sel, total, tailv)
        pltpu.sync_copy(rbuf, rp_hbm)


def _sc_scatter_body(dst_hbm, src_hbm, wstart_hbm, rp_hbm, lh_hbm, sd_hbm, ss_hbm,
                     dstv, srcv, lanehist, postab, wsbuf, rpbuf, posb, sem):
    wid = lax.axis_index("s") * 2 + lax.axis_index("c")
    base = wid * EPW
    pltpu.sync_copy(dst_hbm.at[pl.ds(base, EPW)], dstv)
    pltpu.sync_copy(src_hbm.at[pl.ds(base, EPW)], srcv)
    pltpu.sync_copy(wstart_hbm.at[wid], wsbuf)
    pltpu.sync_copy(rp_hbm.at[pl.ds(0, N_NODES)], rpbuf)
    pltpu.sync_copy(lh_hbm.at[wid], lanehist)
    lane = lax.iota(jnp.int32, NL)

    def ibody(j, c):
        run = rpbuf[pl.ds(j * NL, NL)] + wsbuf[pl.ds(j * NL, NL)]
        for l in range(NL):
            cnt = lanehist[pl.ds(l * N_NODES + j * NL, NL)]
            postab[pl.ds(l * N_NODES + j * NL, NL)] = run
            run = run + cnt
        return c

    lax.fori_loop(0, N_NODES // NL, ibody, 0)

    def sbody(k, c):
        e = lane * EPL + k
        d = plsc.load_gather(dstv, [e])
        a = lane * N_NODES + d
        p = plsc.load_gather(postab, [a])
        plsc.store_scatter(postab, [a], p + 1)
        plsc.store_scatter(posb, [lane, jnp.full((NL,), k, jnp.int32)], p)
        return c

    lax.fori_loop(0, EPL, sbody, 0)

    for l in range(NL):
        cd = pltpu.async_copy(dstv.at[pl.ds(l * EPL, EPL)], sd_hbm.at[posb.at[l]], sem)
        cd.wait()
        cs = pltpu.async_copy(srcv.at[pl.ds(l * EPL, EPL)], ss_hbm.at[posb.at[l]], sem)
        cs.wait()


def _sc_sort(dst, src):
    mesh = plsc.VectorSubcoreMesh(core_axis_name="c", subcore_axis_name="s")
    nolayout = pltpu.CompilerParams(needs_layout_passes=False)
    zeros32 = jnp.zeros((NL * N_NODES,), jnp.int32)
    counts, lh = functools.partial(
        pl.kernel, _sc_hist_body, mesh=mesh, compiler_params=nolayout,
        out_type=(
            jax.ShapeDtypeStruct((N_WORKERS, N_NODES), jnp.int32),
            jax.ShapeDtypeStruct((N_WORKERS, NL * N_NODES), jnp.int32),
        ),
        scratch_types=[
            pltpu.VMEM((EPW,), jnp.int32),
            pltpu.VMEM((NL * N_NODES,), jnp.int32),
            pltpu.VMEM((N_NODES,), jnp.int32),
        ],
    )()(dst, zeros32)
    wstart, totals = functools.partial(
        pl.kernel, _sc_wprefix_body, mesh=mesh, compiler_params=nolayout,
        out_type=(
            jax.ShapeDtypeStruct((N_WORKERS, N_NODES), jnp.int32),
            jax.ShapeDtypeStruct((N_NODES,), jnp.int32),
        ),
        scratch_types=[
            pltpu.VMEM((N_NODES,), jnp.int32),
            pltpu.VMEM((N_NODES,), jnp.int32),
            pltpu.VMEM((N_NODES // N_WORKERS,), jnp.int32),
        ],
    )()(counts)
    rp = functools.partial(
        pl.kernel, _sc_rowptr_body, mesh=mesh, compiler_params=nolayout,
        out_type=jax.ShapeDtypeStruct((N_NODES + 8,), jnp.int32),
        scratch_types=[
            pltpu.VMEM((N_NODES,), jnp.int32),
            pltpu.VMEM((N_NODES + 8,), jnp.int32),
        ],
    )()(totals)
    sd, ssrc = functools.partial(
        pl.kernel, _sc_scatter_body, mesh=mesh, compiler_params=nolayout,
        out_type=(
            jax.ShapeDtypeStruct((N_EDGES,), jnp.int32),
            jax.ShapeDtypeStruct((N_EDGES,), jnp.int32),
        ),
        scratch_types=[
            pltpu.VMEM((EPW,), jnp.int32),
            pltpu.VMEM((EPW,), jnp.int32),
            pltpu.VMEM((NL * N_NODES,), jnp.int32),
            pltpu.VMEM((NL * N_NODES,), jnp.int32),
            pltpu.VMEM((N_NODES,), jnp.int32),
            pltpu.VMEM((N_NODES,), jnp.int32),
            pltpu.VMEM((NL, EPL), jnp.int32),
            pltpu.SemaphoreType.DMA,
        ],
    )()(dst, src, wstart, rp, lh)
    return sd, ssrc, rp


RING = 4


def _sc_gather_body(qt_hbm, ss_hbm, outq_hbm,
                    ss_v, qr0, qr1, qr2, qr3, sq0, sq1, sq2, sq3):
    wid = lax.axis_index("s") * 2 + lax.axis_index("c")
    base = wid * EPW
    pltpu.sync_copy(ss_hbm.at[pl.ds(base, EPW)], ss_v)
    qrb = [qr0, qr1, qr2, qr3]
    sqs = [sq0, sq1, sq2, sq3]

    def fire(j):
        r = j % RING
        idx_s = ss_v.at[pl.ds(j * SUB, SUB)]
        return pltpu.async_copy(qt_hbm.at[idx_s], qrb[r], sqs[r])

    inflight = [fire(j) for j in range(RING)]
    for j in range(N_SUB):
        r = j % RING
        inflight[r].wait()
        off = base + j * SUB
        pltpu.sync_copy(qrb[r], outq_hbm.at[pl.ds(off, SUB)])
        if j + RING < N_SUB:
            inflight[r] = fire(j + RING)


def _gather_q(qtab, ssrc):
    f2 = qtab.shape[1]
    mesh = plsc.VectorSubcoreMesh(core_axis_name="c", subcore_axis_name="s")
    fn = functools.partial(
        pl.kernel,
        _sc_gather_body,
        mesh=mesh,
        out_type=jax.ShapeDtypeStruct((N_EDGES, f2), jnp.float32),
        scratch_types=[
            pltpu.VMEM((EPW,), jnp.int32),
        ]
        + [pltpu.VMEM((SUB, f2), jnp.float32) for _ in range(RING)]
        + [pltpu.SemaphoreType.DMA for _ in range(RING)],
    )()
    return fn(qtab, ssrc)


def _conv_kernel(rp_ref, nf_ref, nl_ref, sd_ref, q_ref, p_ref, w_ref, b_ref,
                 o_ref, h_ref, m_ref, acc_ref):
    t = pl.program_id(0)
    c = o_ref.shape[1]
    f = h_ref.shape[1]
    tbase = t * TILE
    nf = nf_ref[t]
    nl = nl_ref[t]

    @pl.when(t == 0)
    def _init():
        acc_ref[...] = jnp.full(acc_ref.shape, NEG_INF, jnp.float32)
        m_ref[pl.ds(TILE, CH), :] = jnp.full((CH, c), NEG_INF, jnp.float32)

    iota_ch = lax.broadcasted_iota(jnp.int32, (CH, 1), 0)

    sdv = sd_ref[...].reshape(1, TILE)
    riota = lax.broadcasted_iota(jnp.int32, (PWIN, TILE), 0)
    st = jnp.where(riota + nf == sdv, 1.0, 0.0)
    pwin = p_ref[pl.ds(nf, PWIN), :]
    prow = lax.dot_general(
        st, pwin, (((0,), (0,)), ((), ())),
        precision=lax.Precision.HIGHEST,
        preferred_element_type=jnp.float32,
    )
    h_ref[...] = jnp.maximum(prow + q_ref[:, :f], 0.0)

    @pl.when(nl - nf > PWIN - 1)
    def _fb():
        def fb_node(n, carry):
            s = jnp.maximum(rp_ref[n] - tbase, 0)
            e = jnp.minimum(rp_ref[n + 1] - tbase, TILE)
            base0 = jnp.minimum((s // 8) * 8, TILE - CH)
            nch = (e - base0 + CH - 1) // CH

            def fb_chunk(ci, carry2):
                b0 = jnp.minimum(base0 + ci * CH, TILE - CH)
                ridx = b0 + iota_ch
                mask = (ridx >= s) & (ridx < e)
                pn = p_ref[pl.ds(n, 1), :]
                newv = jnp.maximum(pn + q_ref[pl.ds(b0, CH), :f], 0.0)
                h_ref[pl.ds(b0, CH), :] = jnp.where(
                    mask, newv, h_ref[pl.ds(b0, CH), :]
                )
                return carry2

            lax.fori_loop(0, nch, fb_chunk, 0)
            return carry

        lax.fori_loop(nf + PWIN, nl + 1, fb_node, 0)

    m_ref[pl.ds(0, TILE), :] = (
        jnp.dot(h_ref[...], w_ref[...], preferred_element_type=jnp.float32)
        + b_ref[...]
    )

    def node_body(n, carry):
        s = jnp.maximum(rp_ref[n] - tbase, 0)
        e = jnp.minimum(rp_ref[n + 1] - tbase, TILE)
        base0 = (s // 8) * 8
        nch = (e - base0 + CH - 1) // CH

        def chunk_body(ci, red):
            b0 = base0 + ci * CH
            rows = m_ref[pl.ds(b0, CH), :]
            ridx = b0 + iota_ch
            mask = (ridx >= s) & (ridx < e)
            return jnp.maximum(red, jnp.where(mask, rows, NEG_INF))

        red = lax.fori_loop(
            0, nch, chunk_body, jnp.full((CH, c), NEG_INF, jnp.float32)
        )
        rowmax = jnp.max(red, axis=0, keepdims=True)
        acc_ref[pl.ds(n, 1), :] = jnp.maximum(acc_ref[pl.ds(n, 1), :], rowmax)
        return carry

    lax.fori_loop(nf, nl + 1, node_body, 0)

    @pl.when(t == N_TILES - 1)
    def _fin():
        a = acc_ref[...]
        o_ref[...] = jnp.where(jnp.isneginf(a), 0.0, a)


def _edge_conv(rp, nfirst, nlast, sd3, rows_q, ppad, w_t, b_row):
    c = w_t.shape[1]
    f2 = rows_q.shape[1]
    grid_spec = pltpu.PrefetchScalarGridSpec(
        num_scalar_prefetch=3,
        grid=(N_TILES,),
        in_specs=[
            pl.BlockSpec((1, 1, TILE), lambda t, *_: (t, 0, 0)),
            pl.BlockSpec((TILE, f2), lambda t, *_: (t, 0)),
            pl.BlockSpec(ppad.shape, lambda t, *_: (0, 0)),
            pl.BlockSpec(w_t.shape, lambda t, *_: (0, 0)),
            pl.BlockSpec(b_row.shape, lambda t, *_: (0, 0)),
        ],
        out_specs=pl.BlockSpec((N_NODES, c), lambda t, *_: (0, 0)),
        scratch_shapes=[
            pltpu.VMEM((TILE, f2 // 2), jnp.float32),
            pltpu.VMEM((TILE + CH, c), jnp.float32),
            pltpu.VMEM((N_NODES, c), jnp.float32),
        ],
    )
    return pl.pallas_call(
        _conv_kernel,
        grid_spec=grid_spec,
        out_shape=jax.ShapeDtypeStruct((N_NODES, c), jnp.float32),
    )(rp, nfirst, nlast, sd3, rows_q, ppad, w_t, b_row)


def _readout_kernel(h_ref, w_ref, b_ref, o_ref):
    o_ref[...] = (
        lax.dot_general(
            h_ref[...],
            w_ref[...],
            (((0,), (1,)), ((), ())),
            preferred_element_type=jnp.float32,
        )
        + b_ref[...]
    )


def kernel(x, edge_index, W1, b1, W2, b2, W3, b3, W4, b4, Wr, br):
    in_ch = x.shape[1]
    hid = W2.shape[0]
    src = edge_index[0].astype(jnp.int32)
    dst = edge_index[1].astype(jnp.int32)

    sd, ssrc, rp = _sc_sort(dst, src)
    sd2 = sd.reshape(N_TILES, TILE)
    nfirst = sd2[:, 0]
    nlast = sd2[:, -1]
    sd3 = sd.reshape(N_TILES, 1, TILE)

    pp1, qt1 = _project(x, W1[:, :in_ch].T, b1.reshape(1, -1), W1[:, in_ch:].T)
    rq1 = _gather_q(qt1, ssrc)
    h1 = _edge_conv(rp, nfirst, nlast, sd3, rq1, pp1, W2.T, b2.reshape(1, -1))

    pp2, qt2 = _project(h1, W3[:, :hid].T, b3.reshape(1, -1), W3[:, hid:].T)
    rq2 = _gather_q(qt2, ssrc)
    h2 = _edge_conv(rp, nfirst, nlast, sd3, rq2, pp2, W4.T, b4.reshape(1, -1))

    out = pl.pallas_call(
        _readout_kernel,
        out_shape=jax.ShapeDtypeStruct((Wr.shape[0], Wr.shape[0]), jnp.float32),
    )(h2, Wr, br.reshape(1, -1))
    return out


# speedup vs baseline: 1.1600x; 1.0663x over previous
import functools

import jax
import jax.numpy as jnp
from jax import lax
from jax.experimental import pallas as pl
from jax.experimental.pallas import tpu as pltpu
from jax.experimental.pallas import tpu_sc as plsc

N_NODES = 2048
N_EDGES = 65536
TILE = 256
N_TILES = N_EDGES // TILE
CH = 32
PWIN = 64

N_WORKERS = 32
EPW = N_EDGES // N_WORKERS
SUB = 128
N_SUB = EPW // SUB

NEG_INF = float("-inf")


def _proj_kernel(x_ref, wa_ref, bias_ref, wb_ref, pp_ref, qt_ref):
    f = wa_ref.shape[1]
    pp_ref[pl.ds(0, N_NODES), :] = (
        jnp.dot(x_ref[...], wa_ref[...], preferred_element_type=jnp.float32)
        + bias_ref[...]
    )
    pp_ref[pl.ds(N_NODES, TILE), :] = jnp.zeros((TILE, f), jnp.float32)
    qt_ref[:, pl.ds(0, f)] = jnp.dot(
        x_ref[...], wb_ref[...], preferred_element_type=jnp.float32
    )
    qt_ref[:, pl.ds(f, f)] = jnp.zeros((N_NODES, f), jnp.float32)


def _project(x, wa, bias, wb):
    f = wa.shape[1]
    return pl.pallas_call(
        _proj_kernel,
        out_shape=(
            jax.ShapeDtypeStruct((N_NODES + TILE, f), jnp.float32),
            jax.ShapeDtypeStruct((N_NODES, 2 * f), jnp.float32),
        ),
    )(x, wa, bias, wb)


NL = 16
EPL = EPW // NL


def _sc_hist_body(dst_hbm, zeros_hbm, counts_hbm, lh_hbm, dstv, lanehist, stage):
    wid = lax.axis_index("s") * 2 + lax.axis_index("c")
    base = wid * EPW
    pltpu.sync_copy(dst_hbm.at[pl.ds(base, EPW)], dstv)
    pltpu.sync_copy(zeros_hbm, lanehist)
    zero = jnp.zeros((NL,), jnp.int32)
    lane = lax.iota(jnp.int32, NL)

    def hbody(k, c):
        d = plsc.load_gather(dstv, [lane * EPL + k])
        a = lane * N_NODES + d
        cur = plsc.load_gather(lanehist, [a])
        plsc.store_scatter(lanehist, [a], cur + 1)
        return c

    lax.fori_loop(0, EPL, hbody, 0)
    pltpu.sync_copy(lanehist, lh_hbm.at[wid])

    def rbody(v4, c):
        for u in range(4):
            v = v4 * 4 + u
            acc = zero
            for l in range(NL):
                acc = acc + lanehist[pl.ds(l * N_NODES + v * NL, NL)]
            stage[pl.ds(v * NL, NL)] = acc
        return c

    lax.fori_loop(0, N_NODES // NL // 4, rbody, 0)
    pltpu.sync_copy(stage, counts_hbm.at[wid])


def _sc_wprefix_body(counts_hbm, wstart_hbm, totals_hbm, cbuf, obuf, tot):
    wid = lax.axis_index("s") * 2 + lax.axis_index("c")
    npw = N_NODES // N_WORKERS
    c0 = wid * npw
    for w in range(N_WORKERS):
        pltpu.sync_copy(counts_hbm.at[w, pl.ds(c0, npw)], cbuf.at[pl.ds(w * npw, npw)])
    for j in range(npw // NL):
        run = jnp.zeros((NL,), jnp.int32)
        for w in range(N_WORKERS):
            c = cbuf[pl.ds(w * npw + j * NL, NL)]
            obuf[pl.ds(w * npw + j * NL, NL)] = run
            run = run + c
        tot[pl.ds(j * NL, NL)] = run
    for w in range(N_WORKERS):
        pltpu.sync_copy(obuf.at[pl.ds(w * npw, npw)], wstart_hbm.at[w, pl.ds(c0, npw)])
    pltpu.sync_copy(tot, totals_hbm.at[pl.ds(c0, npw)])


def _sc_rowptr_body(totals_hbm, rp_hbm, tbuf, rbuf):
    wid = lax.axis_index("s") * 2 + lax.axis_index("c")

    @pl.when(wid == 0)
    def _():
        pltpu.sync_copy(totals_hbm, tbuf)

        def body(j, carry):
            v = tbuf[pl.ds(j * NL, NL)]
            s = plsc.cumsum(v)
            rbuf[pl.ds(j * NL, NL)] = (s - v) + carry
            return carry + jnp.sum(v)

        total = lax.fori_loop(0, N_NODES // NL, body, jnp.int32(0))
        tailv = rbuf[pl.ds(N_NODES - 8, NL)]
        sel = lax.iota(jnp.int32, NL) >= 8
        rbuf[pl.ds(N_NODES - 8, NL)] = jnp.where(sel, total, tailv)
        pltpu.sync_copy(rbuf, rp_hbm)


def _sc_scatter_body(dst_hbm, src_hbm, wstart_hbm, rp_hbm, lh_hbm, sd_hbm, ss_hbm,
                     dstv, srcv, lanehist, postab, wsbuf, rpbuf, posb, sem):
    wid = lax.axis_index("s") * 2 + lax.axis_index("c")
    base = wid * EPW
    pltpu.sync_copy(dst_hbm.at[pl.ds(base, EPW)], dstv)
    pltpu.sync_copy(src_hbm.at[pl.ds(base, EPW)], srcv)
    pltpu.sync_copy(wstart_hbm.at[wid], wsbuf)
    pltpu.sync_copy(rp_hbm.at[pl.ds(0, N_NODES)], rpbuf)
    pltpu.sync_copy(lh_hbm.at[wid], lanehist)
    lane = lax.iota(jnp.int32, NL)

    def ibody(j, c):
        run = rpbuf[pl.ds(j * NL, NL)] + wsbuf[pl.ds(j * NL, NL)]
        for l in range(NL):
            cnt = lanehist[pl.ds(l * N_NODES + j * NL, NL)]
            postab[pl.ds(l * N_NODES + j * NL, NL)] = run
            run = run + cnt
        return c

    lax.fori_loop(0, N_NODES // NL, ibody, 0)

    def sbody(k, c):
        e = lane * EPL + k
        d = plsc.load_gather(dstv, [e])
        a = lane * N_NODES + d
        p = plsc.load_gather(postab, [a])
        plsc.store_scatter(postab, [a], p + 1)
        plsc.store_scatter(posb, [lane, jnp.full((NL,), k, jnp.int32)], p)
        return c

    lax.fori_loop(0, EPL, sbody, 0)

    for l in range(NL):
        cd = pltpu.async_copy(dstv.at[pl.ds(l * EPL, EPL)], sd_hbm.at[posb.at[l]], sem)
        cd.wait()
        cs = pltpu.async_copy(srcv.at[pl.ds(l * EPL, EPL)], ss_hbm.at[posb.at[l]], sem)
        cs.wait()


def _sc_sort(dst, src):
    mesh = plsc.VectorSubcoreMesh(core_axis_name="c", subcore_axis_name="s")
    nolayout = pltpu.CompilerParams(needs_layout_passes=False)
    zeros32 = jnp.zeros((NL * N_NODES,), jnp.int32)
    counts, lh = functools.partial(
        pl.kernel, _sc_hist_body, mesh=mesh, compiler_params=nolayout,
        out_type=(
            jax.ShapeDtypeStruct((N_WORKERS, N_NODES), jnp.int32),
            jax.ShapeDtypeStruct((N_WORKERS, NL * N_NODES), jnp.int32),
        ),
        scratch_types=[
            pltpu.VMEM((EPW,), jnp.int32),
            pltpu.VMEM((NL * N_NODES,), jnp.int32),
            pltpu.VMEM((N_NODES,), jnp.int32),
        ],
    )()(dst, zeros32)
    wstart, totals = functools.partial(
        pl.kernel, _sc_wprefix_body, mesh=mesh, compiler_params=nolayout,
        out_type=(
            jax.ShapeDtypeStruct((N_WORKERS, N_NODES), jnp.int32),
            jax.ShapeDtypeStruct((N_NODES,), jnp.int32),
        ),
        scratch_types=[
            pltpu.VMEM((N_NODES,), jnp.int32),
            pltpu.VMEM((N_NODES,), jnp.int32),
            pltpu.VMEM((N_NODES // N_WORKERS,), jnp.int32),
        ],
    )()(counts)
    rp = functools.partial(
        pl.kernel, _sc_rowptr_body, mesh=mesh, compiler_params=nolayout,
        out_type=jax.ShapeDtypeStruct((N_NODES + 8,), jnp.int32),
        scratch_types=[
            pltpu.VMEM((N_NODES,), jnp.int32),
            pltpu.VMEM((N_NODES + 8,), jnp.int32),
        ],
    )()(totals)
    sd, ssrc = functools.partial(
        pl.kernel, _sc_scatter_body, mesh=mesh, compiler_params=nolayout,
        out_type=(
            jax.ShapeDtypeStruct((N_EDGES,), jnp.int32),
            jax.ShapeDtypeStruct((N_EDGES,), jnp.int32),
        ),
        scratch_types=[
            pltpu.VMEM((EPW,), jnp.int32),
            pltpu.VMEM((EPW,), jnp.int32),
            pltpu.VMEM((NL * N_NODES,), jnp.int32),
            pltpu.VMEM((NL * N_NODES,), jnp.int32),
            pltpu.VMEM((N_NODES,), jnp.int32),
            pltpu.VMEM((N_NODES,), jnp.int32),
            pltpu.VMEM((NL, EPL), jnp.int32),
            pltpu.SemaphoreType.DMA,
        ],
    )()(dst, src, wstart, rp, lh)
    return sd, ssrc, rp


RING = 4


def _sc_gather_body(qt_hbm, ss_hbm, outq_hbm,
                    ss_v, qr0, qr1, qr2, qr3, sq0, sq1, sq2, sq3):
    wid = lax.axis_index("s") * 2 + lax.axis_index("c")
    base = wid * EPW
    pltpu.sync_copy(ss_hbm.at[pl.ds(base, EPW)], ss_v)
    qrb = [qr0, qr1, qr2, qr3]
    sqs = [sq0, sq1, sq2, sq3]

    def fire(j):
        r = j % RING
        idx_s = ss_v.at[pl.ds(j * SUB, SUB)]
        return pltpu.async_copy(qt_hbm.at[idx_s], qrb[r], sqs[r])

    inflight = [fire(j) for j in range(RING)]
    for j in range(N_SUB):
        r = j % RING
        inflight[r].wait()
        off = base + j * SUB
        pltpu.sync_copy(qrb[r], outq_hbm.at[pl.ds(off, SUB)])
        if j + RING < N_SUB:
            inflight[r] = fire(j + RING)


def _gather_q(qtab, ssrc):
    f2 = qtab.shape[1]
    mesh = plsc.VectorSubcoreMesh(core_axis_name="c", subcore_axis_name="s")
    fn = functools.partial(
        pl.kernel,
        _sc_gather_body,
        mesh=mesh,
        out_type=jax.ShapeDtypeStruct((N_EDGES, f2), jnp.float32),
        scratch_types=[
            pltpu.VMEM((EPW,), jnp.int32),
        ]
        + [pltpu.VMEM((SUB, f2), jnp.float32) for _ in range(RING)]
        + [pltpu.SemaphoreType.DMA for _ in range(RING)],
    )()
    return fn(qtab, ssrc)


def _conv_kernel(rp_ref, nf_ref, nl_ref, sd_ref, q_ref, p_ref, w_ref, b_ref,
                 o_ref, h_ref, m_ref, acc_ref):
    t = pl.program_id(0)
    c = o_ref.shape[1]
    f = h_ref.shape[1]
    tbase = t * TILE
    nf = nf_ref[t]
    nl = nl_ref[t]

    @pl.when(t == 0)
    def _init():
        acc_ref[...] = jnp.full(acc_ref.shape, NEG_INF, jnp.float32)
        m_ref[pl.ds(TILE, CH), :] = jnp.full((CH, c), NEG_INF, jnp.float32)

    iota_ch = lax.broadcasted_iota(jnp.int32, (CH, 1), 0)

    sdv = sd_ref[...].reshape(1, TILE)
    riota = lax.broadcasted_iota(jnp.int32, (PWIN, TILE), 0)
    st = jnp.where(riota + nf == sdv, 1.0, 0.0)
    pwin = p_ref[pl.ds(nf, PWIN), :]
    p_hi = pwin.astype(jnp.bfloat16).astype(jnp.float32)
    p_lo = pwin - p_hi
    dims = (((0,), (0,)), ((), ()))
    prow = lax.dot_general(
        st, p_hi, dims, preferred_element_type=jnp.float32
    ) + lax.dot_general(st, p_lo, dims, preferred_element_type=jnp.float32)
    h_ref[...] = jnp.maximum(prow + q_ref[:, :f], 0.0)

    @pl.when(nl - nf > PWIN - 1)
    def _fb():
        def fb_node(n, carry):
            s = jnp.maximum(rp_ref[n] - tbase, 0)
            e = jnp.minimum(rp_ref[n + 1] - tbase, TILE)
            base0 = jnp.minimum((s // 8) * 8, TILE - CH)
            nch = (e - base0 + CH - 1) // CH

            def fb_chunk(ci, carry2):
                b0 = jnp.minimum(base0 + ci * CH, TILE - CH)
                ridx = b0 + iota_ch
                mask = (ridx >= s) & (ridx < e)
                pn = p_ref[pl.ds(n, 1), :]
                newv = jnp.maximum(pn + q_ref[pl.ds(b0, CH), :f], 0.0)
                h_ref[pl.ds(b0, CH), :] = jnp.where(
                    mask, newv, h_ref[pl.ds(b0, CH), :]
                )
                return carry2

            lax.fori_loop(0, nch, fb_chunk, 0)
            return carry

        lax.fori_loop(nf + PWIN, nl + 1, fb_node, 0)

    m_ref[pl.ds(0, TILE), :] = (
        jnp.dot(h_ref[...], w_ref[...], preferred_element_type=jnp.float32)
        + b_ref[...]
    )

    def node_body(n, carry):
        s = jnp.maximum(rp_ref[n] - tbase, 0)
        e = jnp.minimum(rp_ref[n + 1] - tbase, TILE)
        base0 = (s // 8) * 8
        nch = (e - base0 + CH - 1) // CH

        def chunk_body(ci, red):
            b0 = base0 + ci * CH
            rows = m_ref[pl.ds(b0, CH), :]
            ridx = b0 + iota_ch
            mask = (ridx >= s) & (ridx < e)
            return jnp.maximum(red, jnp.where(mask, rows, NEG_INF))

        red = lax.fori_loop(
            0, nch, chunk_body, jnp.full((CH, c), NEG_INF, jnp.float32)
        )
        rowmax = jnp.max(red, axis=0, keepdims=True)
        acc_ref[pl.ds(n, 1), :] = jnp.maximum(acc_ref[pl.ds(n, 1), :], rowmax)
        return carry

    lax.fori_loop(nf, nl + 1, node_body, 0)

    @pl.when(t == N_TILES - 1)
    def _fin():
        a = acc_ref[...]
        o_ref[...] = jnp.where(jnp.isneginf(a), 0.0, a)


def _edge_conv(rp, nfirst, nlast, sd3, rows_q, ppad, w_t, b_row):
    c = w_t.shape[1]
    f2 = rows_q.shape[1]
    grid_spec = pltpu.PrefetchScalarGridSpec(
        num_scalar_prefetch=3,
        grid=(N_TILES,),
        in_specs=[
            pl.BlockSpec((1, 1, TILE), lambda t, *_: (t, 0, 0)),
            pl.BlockSpec((TILE, f2), lambda t, *_: (t, 0)),
            pl.BlockSpec(ppad.shape, lambda t, *_: (0, 0)),
            pl.BlockSpec(w_t.shape, lambda t, *_: (0, 0)),
            pl.BlockSpec(b_row.shape, lambda t, *_: (0, 0)),
        ],
        out_specs=pl.BlockSpec((N_NODES, c), lambda t, *_: (0, 0)),
        scratch_shapes=[
            pltpu.VMEM((TILE, f2 // 2), jnp.float32),
            pltpu.VMEM((TILE + CH, c), jnp.float32),
            pltpu.VMEM((N_NODES, c), jnp.float32),
        ],
    )
    return pl.pallas_call(
        _conv_kernel,
        grid_spec=grid_spec,
        out_shape=jax.ShapeDtypeStruct((N_NODES, c), jnp.float32),
    )(rp, nfirst, nlast, sd3, rows_q, ppad, w_t, b_row)


def _readout_kernel(h_ref, w_ref, b_ref, o_ref):
    o_ref[...] = (
        lax.dot_general(
            h_ref[...],
            w_ref[...],
            (((0,), (1,)), ((), ())),
            preferred_element_type=jnp.float32,
        )
        + b_ref[...]
    )


def kernel(x, edge_index, W1, b1, W2, b2, W3, b3, W4, b4, Wr, br):
    in_ch = x.shape[1]
    hid = W2.shape[0]
    src = edge_index[0].astype(jnp.int32)
    dst = edge_index[1].astype(jnp.int32)

    sd, ssrc, rp = _sc_sort(dst, src)
    sd2 = sd.reshape(N_TILES, TILE)
    nfirst = sd2[:, 0]
    nlast = sd2[:, -1]
    sd3 = sd.reshape(N_TILES, 1, TILE)

    pp1, qt1 = _project(x, W1[:, :in_ch].T, b1.reshape(1, -1), W1[:, in_ch:].T)
    rq1 = _gather_q(qt1, ssrc)
    h1 = _edge_conv(rp, nfirst, nlast, sd3, rq1, pp1, W2.T, b2.reshape(1, -1))

    pp2, qt2 = _project(h1, W3[:, :hid].T, b3.reshape(1, -1), W3[:, hid:].T)
    rq2 = _gather_q(qt2, ssrc)
    h2 = _edge_conv(rp, nfirst, nlast, sd3, rq2, pp2, W4.T, b4.reshape(1, -1))

    out = pl.pallas_call(
        _readout_kernel,
        out_shape=jax.ShapeDtypeStruct((Wr.shape[0], Wr.shape[0]), jnp.float32),
    )(h2, Wr, br.reshape(1, -1))
    return out
